# speedup vs baseline: 1.2754x; 1.2754x over previous
import functools

import jax
import jax.numpy as jnp
from jax import lax
from jax.experimental import pallas as pl
from jax.experimental.pallas import tpu as pltpu
from jax.experimental.pallas import tpu_sc as plsc

PREC = 12
KERN = 64
D = PREC * KERN


def _conv_transform(x2d, w0, w1, w2, bias, apply_act, bm=1200):
    M, C = x2d.shape

    def body(x_ref, w0_ref, w1_ref, w2_ref, b_ref, g0_ref, g1_ref, g2_ref):
        x = x_ref[...]
        if apply_act:
            x = jnp.maximum(x + b_ref[...], 0.0)
        w_id = lax.broadcasted_iota(jnp.int32, (bm, 1), 0) % PREC
        zrow = jnp.zeros((1, KERN), jnp.float32)
        for w_ref, g_ref in ((w0_ref, g0_ref), (w1_ref, g1_ref), (w2_ref, g2_ref)):
            p = jnp.dot(x, w_ref[...], preferred_element_type=jnp.float32)
            a0 = p[:, 0:KERN]
            a1 = p[:, KERN:2 * KERN]
            a2 = p[:, 2 * KERN:]
            a0s = jnp.concatenate([zrow, a0[:-1]], axis=0)
            a2s = jnp.concatenate([a2[1:], zrow], axis=0)
            g_ref[...] = (a1
                          + jnp.where(w_id != 0, a0s, 0.0)
                          + jnp.where(w_id != PREC - 1, a2s, 0.0))

    out = jax.ShapeDtypeStruct((M, KERN), jnp.float32)
    return pl.pallas_call(
        body,
        grid=(M // bm,),
        in_specs=[
            pl.BlockSpec((bm, C), lambda i: (i, 0)),
            pl.BlockSpec((C, 3 * KERN), lambda i: (0, 0)),
            pl.BlockSpec((C, 3 * KERN), lambda i: (0, 0)),
            pl.BlockSpec((C, 3 * KERN), lambda i: (0, 0)),
            pl.BlockSpec((1, C), lambda i: (0, 0)),
        ],
        out_specs=[pl.BlockSpec((bm, KERN), lambda i: (i, 0))] * 3,
        out_shape=[out, out, out],
    )(x2d, w0, w1, w2, bias)


def _split_conv_w(W):
    C = W.shape[1] // 3
    Wr = W.reshape(3, 3, C, KERN)
    return [Wr[:, d].transpose(1, 0, 2).reshape(C, 3 * KERN) for d in range(3)]


def _gather_sum(g0, g1, g2, i0, i1, i2):
    N = i0.shape[0]
    info = plsc.get_sparse_core_info()
    NC, NS, L = info.num_cores, info.num_subcores, info.num_lanes
    NW = NC * NS
    R = 40
    CH = N // R

    mesh = plsc.VectorSubcoreMesh(core_axis_name="c", subcore_axis_name="s")

    @functools.partial(
        pl.kernel,
        mesh=mesh,
        out_type=jax.ShapeDtypeStruct((N, D), jnp.float32),
        scratch_types=[
            pltpu.VMEM((R,), jnp.int32),
            pltpu.VMEM((R,), jnp.int32),
            pltpu.VMEM((R,), jnp.int32),
            pltpu.VMEM((R, D), jnp.float32),
            pltpu.VMEM((R, D), jnp.float32),
            pltpu.VMEM((R, D), jnp.float32),
            pltpu.SemaphoreType.DMA,
        ],
    )
    def k(g0_h, g1_h, g2_h, i0_h, i1_h, i2_h, out_h,
          ix0, ix1, ix2, b0, b1, b2, sem):
        wid = lax.axis_index("s") * NC + lax.axis_index("c")
        c_lo = wid * CH // NW
        c_hi = (wid + 1) * CH // NW

        def chunk(ci, carry):
            base = ci * R
            pltpu.sync_copy(i0_h.at[pl.ds(base, R)], ix0)
            pltpu.sync_copy(i1_h.at[pl.ds(base, R)], ix1)
            pltpu.sync_copy(i2_h.at[pl.ds(base, R)], ix2)
            cp0 = pltpu.async_copy(g0_h.at[ix0], b0, sem)
            cp1 = pltpu.async_copy(g1_h.at[ix1], b1, sem)
            cp2 = pltpu.async_copy(g2_h.at[ix2], b2, sem)
            cp0.wait()
            cp1.wait()
            cp2.wait()

            def add_row(r, carry2):
                for j in range(D // L):
                    sl = pl.ds(j * L, L)
                    b0[r, sl] = b0[r, sl] + b1[r, sl] + b2[r, sl]
                return carry2

            lax.fori_loop(0, R, add_row, 0)
            pltpu.sync_copy(b0, out_h.at[pl.ds(base, R)])
            return carry

        lax.fori_loop(c_lo, c_hi, chunk, 0)

    return k(g0, g1, g2, i0, i1, i2)


def _dense_head(y2, b2tile, dW0, db0, dW1, db1, dW2, db2,
                ln_gamma, ln_beta, head_W, head_b, mask, bn=400):
    N = y2.shape[0]
    H = dW0.shape[1]
    BINS = head_W.shape[1]

    def body(y_ref, bt_ref, w0_ref, b0_ref, w1_ref, b1_ref, w2_ref, b2_ref,
             g_ref, be_ref, hw_ref, hb_ref, m_ref, o_ref):
        h = jnp.maximum(y_ref[...] + bt_ref[...], 0.0)
        z = jnp.maximum(jnp.dot(h, w0_ref[...],
                                preferred_element_type=jnp.float32) + b0_ref[...], 0.0)
        z = jnp.maximum(jnp.dot(z, w1_ref[...],
                                preferred_element_type=jnp.float32) + b1_ref[...], 0.0)
        z = jnp.maximum(jnp.dot(z, w2_ref[...],
                                preferred_element_type=jnp.float32) + b2_ref[...], 0.0)
        mu = jnp.mean(z, axis=-1, keepdims=True)
        zc = z - mu
        var = jnp.mean(zc * zc, axis=-1, keepdims=True)
        xn = zc * lax.rsqrt(var + 1e-3) * g_ref[...] + be_ref[...]
        logits = jnp.dot(xn, hw_ref[...],
                         preferred_element_type=jnp.float32) + hb_ref[...]
        mx = jnp.max(logits, axis=-1, keepdims=True)
        e = jnp.exp(logits - mx)
        p = e / jnp.sum(e, axis=-1, keepdims=True)
        o_ref[...] = p * m_ref[...]

    def full(shape):
        return pl.BlockSpec(shape, lambda i: (0, 0))

    return pl.pallas_call(
        body,
        grid=(N // bn,),
        in_specs=[
            pl.BlockSpec((bn, D), lambda i: (i, 0)),
            full((1, D)),
            full((D, H)), full((1, H)),
            full((H, H)), full((1, H)),
            full((H, H)), full((1, H)),
            full((1, H)), full((1, H)),
            full((H, BINS)), full((1, BINS)),
            pl.BlockSpec((bn, BINS), lambda i: (i, 0)),
        ],
        out_specs=pl.BlockSpec((bn, BINS), lambda i: (i, 0)),
        out_shape=jax.ShapeDtypeStruct((N, BINS), jnp.float32),
    )(y2, b2tile, dW0, db0, dW1, db1, dW2, db2,
      ln_gamma, ln_beta, head_W, head_b, mask)


def kernel(features, index, mask,
           conv_W0, conv_b0, conv_W1, conv_b1, conv_W2, conv_b2,
           dense_W0, dense_b0, dense_W1, dense_b1, dense_W2, dense_b2,
           ln_gamma, ln_beta, head_W, head_b):
    N = features.shape[0]
    i0, i1, i2 = index[:, 0], index[:, 1], index[:, 2]

    x2d = features.reshape(N * PREC, -1)
    C0 = x2d.shape[1]
    w0s = _split_conv_w(conv_W0)
    g = _conv_transform(x2d, *w0s, jnp.zeros((1, C0), jnp.float32),
                        apply_act=False)
    y = _gather_sum(g[0].reshape(N, D), g[1].reshape(N, D),
                    g[2].reshape(N, D), i0, i1, i2)

    for W, b_prev in ((conv_W1, conv_b0), (conv_W2, conv_b1)):
        ws = _split_conv_w(W)
        g = _conv_transform(y.reshape(N * PREC, KERN), *ws,
                            b_prev.reshape(1, KERN), apply_act=True)
        y = _gather_sum(g[0].reshape(N, D), g[1].reshape(N, D),
                        g[2].reshape(N, D), i0, i1, i2)

    b2tile = jnp.tile(conv_b2, PREC).reshape(1, D)
    return _dense_head(
        y, b2tile,
        dense_W0, dense_b0.reshape(1, -1),
        dense_W1, dense_b1.reshape(1, -1),
        dense_W2, dense_b2.reshape(1, -1),
        ln_gamma.reshape(1, -1), ln_beta.reshape(1, -1),
        head_W, head_b.reshape(1, -1), mask)


# speedup vs baseline: 1.5526x; 1.2174x over previous
import functools

import jax
import jax.numpy as jnp
from jax import lax
from jax.experimental import pallas as pl
from jax.experimental.pallas import tpu as pltpu
from jax.experimental.pallas import tpu_sc as plsc

PREC = 12
KERN = 64
D = PREC * KERN


def _conv_transform(x2d, w0, w1, w2, bias, apply_act, bm=4800):
    M, C = x2d.shape

    def body(x_ref, w0_ref, w1_ref, w2_ref, b_ref, g0_ref, g1_ref, g2_ref):
        x = x_ref[...]
        if apply_act:
            x = jnp.maximum(x + b_ref[...], 0.0)
        w_id = lax.broadcasted_iota(jnp.int32, (bm, 1), 0) % PREC
        zrow = jnp.zeros((1, C), jnp.float32)
        xp = jnp.where(w_id != 0,
                       jnp.concatenate([zrow, x[:-1]], axis=0), 0.0)
        xn = jnp.where(w_id != PREC - 1,
                       jnp.concatenate([x[1:], zrow], axis=0), 0.0)
        xcat = jnp.concatenate([xp, x, xn], axis=1)
        for w_ref, g_ref in ((w0_ref, g0_ref), (w1_ref, g1_ref), (w2_ref, g2_ref)):
            g_ref[...] = jnp.dot(xcat, w_ref[...],
                                 preferred_element_type=jnp.float32)

    out = jax.ShapeDtypeStruct((M, KERN), jnp.float32)
    return pl.pallas_call(
        body,
        grid=(M // bm,),
        in_specs=[
            pl.BlockSpec((bm, C), lambda i: (i, 0)),
            pl.BlockSpec((3 * C, KERN), lambda i: (0, 0)),
            pl.BlockSpec((3 * C, KERN), lambda i: (0, 0)),
            pl.BlockSpec((3 * C, KERN), lambda i: (0, 0)),
            pl.BlockSpec((1, C), lambda i: (0, 0)),
        ],
        out_specs=[pl.BlockSpec((bm, KERN), lambda i: (i, 0))] * 3,
        out_shape=[out, out, out],
    )(x2d, w0, w1, w2, bias)


def _split_conv_w(W):
    C = W.shape[1] // 3
    Wr = W.reshape(3, 3, C, KERN)
    return [Wr[:, d].reshape(3 * C, KERN) for d in range(3)]


def _gather_sum(g0, g1, g2, i0, i1, i2):
    N = i0.shape[0]
    info = plsc.get_sparse_core_info()
    NC, NS, L = info.num_cores, info.num_subcores, info.num_lanes
    NW = NC * NS
    R = 40
    CH = N // R

    mesh = plsc.VectorSubcoreMesh(core_axis_name="c", subcore_axis_name="s")

    @functools.partial(
        pl.kernel,
        mesh=mesh,
        out_type=jax.ShapeDtypeStruct((N, D), jnp.float32),
        scratch_types=[
            pltpu.VMEM((R,), jnp.int32),
            pltpu.VMEM((R,), jnp.int32),
            pltpu.VMEM((R,), jnp.int32),
            pltpu.VMEM((R, D), jnp.float32),
            pltpu.VMEM((R, D), jnp.float32),
            pltpu.VMEM((R, D), jnp.float32),
            pltpu.SemaphoreType.DMA,
        ],
    )
    def k(g0_h, g1_h, g2_h, i0_h, i1_h, i2_h, out_h,
          ix0, ix1, ix2, b0, b1, b2, sem):
        wid = lax.axis_index("s") * NC + lax.axis_index("c")
        c_lo = wid * CH // NW
        c_hi = (wid + 1) * CH // NW

        def chunk(ci, carry):
            base = ci * R
            pltpu.sync_copy(i0_h.at[pl.ds(base, R)], ix0)
            pltpu.sync_copy(i1_h.at[pl.ds(base, R)], ix1)
            pltpu.sync_copy(i2_h.at[pl.ds(base, R)], ix2)
            cp0 = pltpu.async_copy(g0_h.at[ix0], b0, sem)
            cp1 = pltpu.async_copy(g1_h.at[ix1], b1, sem)
            cp2 = pltpu.async_copy(g2_h.at[ix2], b2, sem)
            cp0.wait()
            cp1.wait()
            cp2.wait()

            def add_row(r, carry2):
                for j in range(D // L):
                    sl = pl.ds(j * L, L)
                    b0[r, sl] = b0[r, sl] + b1[r, sl] + b2[r, sl]
                return carry2

            lax.fori_loop(0, R, add_row, 0)
            pltpu.sync_copy(b0, out_h.at[pl.ds(base, R)])
            return carry

        lax.fori_loop(c_lo, c_hi, chunk, 0)

    return k(g0, g1, g2, i0, i1, i2)


def _dense_head(y2, b2tile, dW0, db0, dW1, db1, dW2, db2,
                ln_gamma, ln_beta, head_W, head_b, mask, bn=400):
    N = y2.shape[0]
    H = dW0.shape[1]
    BINS = head_W.shape[1]

    def body(y_ref, bt_ref, w0_ref, b0_ref, w1_ref, b1_ref, w2_ref, b2_ref,
             g_ref, be_ref, hw_ref, hb_ref, m_ref, o_ref):
        h = jnp.maximum(y_ref[...] + bt_ref[...], 0.0)
        z = jnp.maximum(jnp.dot(h, w0_ref[...],
                                preferred_element_type=jnp.float32) + b0_ref[...], 0.0)
        z = jnp.maximum(jnp.dot(z, w1_ref[...],
                                preferred_element_type=jnp.float32) + b1_ref[...], 0.0)
        z = jnp.maximum(jnp.dot(z, w2_ref[...],
                                preferred_element_type=jnp.float32) + b2_ref[...], 0.0)
        mu = jnp.mean(z, axis=-1, keepdims=True)
        zc = z - mu
        var = jnp.mean(zc * zc, axis=-1, keepdims=True)
        xn = zc * lax.rsqrt(var + 1e-3) * g_ref[...] + be_ref[...]
        logits = jnp.dot(xn, hw_ref[...],
                         preferred_element_type=jnp.float32) + hb_ref[...]
        mx = jnp.max(logits, axis=-1, keepdims=True)
        e = jnp.exp(logits - mx)
        p = e / jnp.sum(e, axis=-1, keepdims=True)
        o_ref[...] = p * m_ref[...]

    def full(shape):
        return pl.BlockSpec(shape, lambda i: (0, 0))

    return pl.pallas_call(
        body,
        grid=(N // bn,),
        in_specs=[
            pl.BlockSpec((bn, D), lambda i: (i, 0)),
            full((1, D)),
            full((D, H)), full((1, H)),
            full((H, H)), full((1, H)),
            full((H, H)), full((1, H)),
            full((1, H)), full((1, H)),
            full((H, BINS)), full((1, BINS)),
            pl.BlockSpec((bn, BINS), lambda i: (i, 0)),
        ],
        out_specs=pl.BlockSpec((bn, BINS), lambda i: (i, 0)),
        out_shape=jax.ShapeDtypeStruct((N, BINS), jnp.float32),
    )(y2, b2tile, dW0, db0, dW1, db1, dW2, db2,
      ln_gamma, ln_beta, head_W, head_b, mask)


def kernel(features, index, mask,
           conv_W0, conv_b0, conv_W1, conv_b1, conv_W2, conv_b2,
           dense_W0, dense_b0, dense_W1, dense_b1, dense_W2, dense_b2,
           ln_gamma, ln_beta, head_W, head_b):
    N = features.shape[0]
    i0, i1, i2 = index[:, 0], index[:, 1], index[:, 2]

    x2d = features.reshape(N * PREC, -1)
    C0 = x2d.shape[1]
    w0s = _split_conv_w(conv_W0)
    g = _conv_transform(x2d, *w0s, jnp.zeros((1, C0), jnp.float32),
                        apply_act=False)
    y = _gather_sum(g[0].reshape(N, D), g[1].reshape(N, D),
                    g[2].reshape(N, D), i0, i1, i2)

    for W, b_prev in ((conv_W1, conv_b0), (conv_W2, conv_b1)):
        ws = _split_conv_w(W)
        g = _conv_transform(y.reshape(N * PREC, KERN), *ws,
                            b_prev.reshape(1, KERN), apply_act=True)
        y = _gather_sum(g[0].reshape(N, D), g[1].reshape(N, D),
                        g[2].reshape(N, D), i0, i1, i2)

    b2tile = jnp.tile(conv_b2, PREC).reshape(1, D)
    return _dense_head(
        y, b2tile,
        dense_W0, dense_b0.reshape(1, -1),
        dense_W1, dense_b1.reshape(1, -1),
        dense_W2, dense_b2.reshape(1, -1),
        ln_gamma.reshape(1, -1), ln_beta.reshape(1, -1),
        head_W, head_b.reshape(1, -1), mask)
